# baseline (speedup 1.0000x reference)
import functools

import jax
import jax.numpy as jnp
from jax import lax
from jax.experimental import pallas as pl

H = 32
EDGE_BLOCK = 8000


def _dot(a, b):
    return jax.lax.dot_general(a, b, (((1,), (0,)), ((), ())),
                               precision=jax.lax.Precision.HIGHEST,
                               preferred_element_type=jnp.float32)


def _edge_mlp_body(pre_ref, b1_ref, w2t_ref, b2_ref, out_ref):
    z = jnp.maximum(pre_ref[...] + b1_ref[...], 0.0)
    out_ref[...] = _dot(z, w2t_ref[...]) + b2_ref[...]


def _edge_mlp(pre, b1, W2, b2):
    E = pre.shape[0]
    grid = E // EDGE_BLOCK
    return pl.pallas_call(
        _edge_mlp_body,
        grid=(grid,),
        in_specs=[
            pl.BlockSpec((EDGE_BLOCK, H), lambda i: (i, 0)),
            pl.BlockSpec((1, H), lambda i: (0, 0)),
            pl.BlockSpec((H, H), lambda i: (0, 0)),
            pl.BlockSpec((1, H), lambda i: (0, 0)),
        ],
        out_specs=pl.BlockSpec((EDGE_BLOCK, H), lambda i: (i, 0)),
        out_shape=jax.ShapeDtypeStruct((E, H), jnp.float32),
    )(pre, b1.reshape(1, H), W2.T, b2.reshape(1, H))


def _edge_mlp_y_body(pre_ref, y_ref, f4t_ref, b1_ref, w2t_ref, b2_ref, out_ref):
    z = pre_ref[...] + _dot(y_ref[...], f4t_ref[...])
    z = jnp.maximum(z + b1_ref[...], 0.0)
    out_ref[...] = _dot(z, w2t_ref[...]) + b2_ref[...]


def _edge_mlp_y(pre, y, F4, b1, W2, b2):
    E = pre.shape[0]
    grid = E // EDGE_BLOCK
    return pl.pallas_call(
        _edge_mlp_y_body,
        grid=(grid,),
        in_specs=[
            pl.BlockSpec((EDGE_BLOCK, H), lambda i: (i, 0)),
            pl.BlockSpec((EDGE_BLOCK, H), lambda i: (i, 0)),
            pl.BlockSpec((H, H), lambda i: (0, 0)),
            pl.BlockSpec((1, H), lambda i: (0, 0)),
            pl.BlockSpec((H, H), lambda i: (0, 0)),
            pl.BlockSpec((1, H), lambda i: (0, 0)),
        ],
        out_specs=pl.BlockSpec((EDGE_BLOCK, H), lambda i: (i, 0)),
        out_shape=jax.ShapeDtypeStruct((E, H), jnp.float32),
    )(pre, y, F4.T, b1.reshape(1, H), W2.T, b2.reshape(1, H))


def kernel(v, edge_index, loop, labels,
           hx_W1, hx_b1, hx_W2, hx_b2,
           hy_W1, hy_b1, hy_W2, hy_b2,
           fx_W1, fx_b1, fx_W2, fx_b2,
           fy_W1, fy_b1, fy_W2, fy_b2,
           feta_W1, feta_b1, feta_W2, feta_b2, feta_W3):
    n, C = v.shape
    vcat = jnp.concatenate([v, labels], axis=-1)
    mask = (labels[:, 1] == 1).astype(vcat.dtype)
    goal = jnp.sum(vcat * mask[:, None], axis=0, keepdims=True)

    H1, H2, H3, H4 = jnp.split(hx_W1, 4, axis=1)
    dd = (vcat - goal) ** 2
    x_pre = vcat @ (H1 + H3).T + dd @ H4.T + (goal @ (H2 - H3).T + hx_b1)
    x = jnp.maximum(x_pre, 0.0) @ hx_W2.T + hx_b2

    src = edge_index[0]
    dst = edge_index[1]

    Y1, Y2, Y3 = jnp.split(hy_W1, 3, axis=1)
    A = vcat @ (Y1 + Y2).T
    B = vcat @ (Y3 - Y1).T
    y = _edge_mlp(A[dst] + B[src], hy_b1, hy_W2, hy_b2)

    F1, F2, F3, F4 = jnp.split(fx_W1, 4, axis=1)
    G1, G2, G3 = jnp.split(fy_W1, 3, axis=1)

    def body(_, carry):
        x, y = carry
        P = x @ (F1 + F2).T
        Q = x @ (F3 - F1).T
        vals = _edge_mlp_y(P[src] + Q[dst], y, F4, fx_b1, fx_W2, fx_b2)
        agg = jax.ops.segment_max(vals, dst, num_segments=n)
        agg = jnp.where(jnp.isneginf(agg), 0.0, agg)
        x = jnp.maximum(x, agg)
        Rp = x @ (G1 + G2).T
        S = x @ (G3 - G1).T
        y = jnp.maximum(y, _edge_mlp(Rp[dst] + S[src], fy_b1, fy_W2, fy_b2))
        return (x, y)

    x, y = lax.fori_loop(0, loop, body, (x, y))

    h = jnp.maximum(x @ feta_W1.T + feta_b1, 0.0)
    h = jnp.maximum(h @ feta_W2.T + feta_b2, 0.0)
    return h @ feta_W3.T


# speedup vs baseline: 2.4076x; 2.4076x over previous
import functools

import jax
import jax.numpy as jnp
from jax import lax
from jax.experimental import pallas as pl
from jax.experimental.pallas import tpu as pltpu
from jax.experimental.pallas import tpu_sc as plsc

H = 32
EDGE_BLOCK = 8000
NW = 32
SC_BLK = 1000


def _sc_gather_add(ptab, qtab, src, dst):
    E = src.shape[0]
    Hf = ptab.shape[1]
    per_w = E // NW
    nblk = per_w // SC_BLK
    mesh = plsc.VectorSubcoreMesh(core_axis_name="c", subcore_axis_name="s")

    @functools.partial(
        pl.kernel,
        mesh=mesh,
        compiler_params=pltpu.CompilerParams(use_tc_tiling_on_sc=False),
        out_type=jax.ShapeDtypeStruct((E, Hf), jnp.float32),
        scratch_types=[
            pltpu.VMEM((SC_BLK,), jnp.int32),
            pltpu.VMEM((SC_BLK,), jnp.int32),
            pltpu.VMEM((SC_BLK, Hf), jnp.float32),
            pltpu.VMEM((SC_BLK, Hf), jnp.float32),
            pltpu.SemaphoreType.DMA,
            pltpu.SemaphoreType.DMA,
        ],
    )
    def k(p_hbm, q_hbm, src_hbm, dst_hbm, out_hbm, sidx, didx, bufp, bufq, sem1, sem2):
        wid = lax.axis_index("s") * 2 + lax.axis_index("c")
        base = wid * per_w

        def blk_body(j, carry):
            off = base + j * SC_BLK
            pltpu.sync_copy(src_hbm.at[pl.ds(off, SC_BLK)], sidx)
            pltpu.sync_copy(dst_hbm.at[pl.ds(off, SC_BLK)], didx)
            cp1 = pltpu.async_copy(p_hbm.at[sidx], bufp, sem1)
            cp2 = pltpu.async_copy(q_hbm.at[didx], bufq, sem2)
            cp1.wait()
            cp2.wait()

            def row_body(i, c):
                a = bufp[i, pl.ds(0, 16)] + bufq[i, pl.ds(0, 16)]
                b = bufp[i, pl.ds(16, 16)] + bufq[i, pl.ds(16, 16)]
                bufp[i, pl.ds(0, 16)] = a
                bufp[i, pl.ds(16, 16)] = b
                return c

            lax.fori_loop(0, SC_BLK, row_body, 0)
            pltpu.sync_copy(bufp, out_hbm.at[pl.ds(off, SC_BLK)])
            return carry

        lax.fori_loop(0, nblk, blk_body, 0)

    return k(ptab, qtab, src, dst)


def _dot(a, b):
    return jax.lax.dot_general(a, b, (((1,), (0,)), ((), ())),
                               precision=jax.lax.Precision.HIGHEST,
                               preferred_element_type=jnp.float32)


def _edge_mlp_body(pre_ref, b1_ref, w2t_ref, b2_ref, out_ref):
    z = jnp.maximum(pre_ref[...] + b1_ref[...], 0.0)
    out_ref[...] = _dot(z, w2t_ref[...]) + b2_ref[...]


def _edge_mlp(pre, b1, W2, b2):
    E = pre.shape[0]
    grid = E // EDGE_BLOCK
    return pl.pallas_call(
        _edge_mlp_body,
        grid=(grid,),
        in_specs=[
            pl.BlockSpec((EDGE_BLOCK, H), lambda i: (i, 0)),
            pl.BlockSpec((1, H), lambda i: (0, 0)),
            pl.BlockSpec((H, H), lambda i: (0, 0)),
            pl.BlockSpec((1, H), lambda i: (0, 0)),
        ],
        out_specs=pl.BlockSpec((EDGE_BLOCK, H), lambda i: (i, 0)),
        out_shape=jax.ShapeDtypeStruct((E, H), jnp.float32),
    )(pre, b1.reshape(1, H), W2.T, b2.reshape(1, H))


def _edge_mlp_y_body(pre_ref, y_ref, f4t_ref, b1_ref, w2t_ref, b2_ref, out_ref):
    z = pre_ref[...] + _dot(y_ref[...], f4t_ref[...])
    z = jnp.maximum(z + b1_ref[...], 0.0)
    out_ref[...] = _dot(z, w2t_ref[...]) + b2_ref[...]


def _edge_mlp_y(pre, y, F4, b1, W2, b2):
    E = pre.shape[0]
    grid = E // EDGE_BLOCK
    return pl.pallas_call(
        _edge_mlp_y_body,
        grid=(grid,),
        in_specs=[
            pl.BlockSpec((EDGE_BLOCK, H), lambda i: (i, 0)),
            pl.BlockSpec((EDGE_BLOCK, H), lambda i: (i, 0)),
            pl.BlockSpec((H, H), lambda i: (0, 0)),
            pl.BlockSpec((1, H), lambda i: (0, 0)),
            pl.BlockSpec((H, H), lambda i: (0, 0)),
            pl.BlockSpec((1, H), lambda i: (0, 0)),
        ],
        out_specs=pl.BlockSpec((EDGE_BLOCK, H), lambda i: (i, 0)),
        out_shape=jax.ShapeDtypeStruct((E, H), jnp.float32),
    )(pre, y, F4.T, b1.reshape(1, H), W2.T, b2.reshape(1, H))


def kernel(v, edge_index, loop, labels,
           hx_W1, hx_b1, hx_W2, hx_b2,
           hy_W1, hy_b1, hy_W2, hy_b2,
           fx_W1, fx_b1, fx_W2, fx_b2,
           fy_W1, fy_b1, fy_W2, fy_b2,
           feta_W1, feta_b1, feta_W2, feta_b2, feta_W3):
    n, C = v.shape
    vcat = jnp.concatenate([v, labels], axis=-1)
    mask = (labels[:, 1] == 1).astype(vcat.dtype)
    goal = jnp.sum(vcat * mask[:, None], axis=0, keepdims=True)

    H1, H2, H3, H4 = jnp.split(hx_W1, 4, axis=1)
    dd = (vcat - goal) ** 2
    x_pre = vcat @ (H1 + H3).T + dd @ H4.T + (goal @ (H2 - H3).T + hx_b1)
    x = jnp.maximum(x_pre, 0.0) @ hx_W2.T + hx_b2

    src = edge_index[0]
    dst = edge_index[1]

    Y1, Y2, Y3 = jnp.split(hy_W1, 3, axis=1)
    A = vcat @ (Y1 + Y2).T
    B = vcat @ (Y3 - Y1).T
    y = _edge_mlp(_sc_gather_add(B, A, src, dst), hy_b1, hy_W2, hy_b2)

    F1, F2, F3, F4 = jnp.split(fx_W1, 4, axis=1)
    G1, G2, G3 = jnp.split(fy_W1, 3, axis=1)

    def body(_, carry):
        x, y = carry
        P = x @ (F1 + F2).T
        Q = x @ (F3 - F1).T
        vals = _edge_mlp_y(_sc_gather_add(P, Q, src, dst), y, F4, fx_b1, fx_W2, fx_b2)
        agg = jax.ops.segment_max(vals, dst, num_segments=n)
        agg = jnp.where(jnp.isneginf(agg), 0.0, agg)
        x = jnp.maximum(x, agg)
        Rp = x @ (G1 + G2).T
        S = x @ (G3 - G1).T
        y = jnp.maximum(y, _edge_mlp(_sc_gather_add(S, Rp, src, dst), fy_b1, fy_W2, fy_b2))
        return (x, y)

    x, y = lax.fori_loop(0, loop, body, (x, y))

    h = jnp.maximum(x @ feta_W1.T + feta_b1, 0.0)
    h = jnp.maximum(h @ feta_W2.T + feta_b2, 0.0)
    return h @ feta_W3.T
